# baseline (speedup 1.0000x reference)
import jax
import jax.numpy as jnp
from jax import lax
from jax.experimental import pallas as pl
from jax.experimental.pallas import tpu as pltpu
from jax.experimental.pallas import tpu_sc as plsc

_NC = 2
_NS = 16
_CH = 128


def _make_sc_agg(n_pad, d, e_pad):
    rows_per_tile = n_pad // _NS
    ept = e_pad // (_NC * _NS)
    nch = ept // _CH
    assert ept % (2 * _CH) == 0 and rows_per_tile % 8 == 0

    def body(h_hbm, src_hbm, dst_hbm, zrow_hbm, p_hbm,
             src_c0, src_c1, dst_c0, dst_c1, buf0, buf1, agg_sh, sem0, sem1):
        c = lax.axis_index("c")
        s = lax.axis_index("s")
        t = c * _NS + s
        r0 = s * rows_per_tile
        base = t * ept
        pltpu.sync_copy(zrow_hbm, agg_sh.at[pl.ds(r0, rows_per_tile)])
        plsc.subcore_barrier()

        def fetch(i, sref, dref):
            pltpu.sync_copy(src_hbm.at[pl.ds(base + i * _CH, _CH)], sref)
            pltpu.sync_copy(dst_hbm.at[pl.ds(base + i * _CH, _CH)], dref)

        def step(k, carry):
            i = 2 * k
            fetch(i, src_c0, dst_c0)
            pltpu.sync_copy(h_hbm.at[src_c0], buf0)
            pltpu.sync_copy(buf0, agg_sh.at[dst_c0], add=True)
            fetch(i + 1, src_c1, dst_c1)
            pltpu.sync_copy(h_hbm.at[src_c1], buf1)
            pltpu.sync_copy(buf1, agg_sh.at[dst_c1], add=True)
            return carry

        lax.fori_loop(0, nch // 2, step, 0)

        plsc.subcore_barrier()
        pltpu.sync_copy(agg_sh.at[pl.ds(r0, rows_per_tile)],
                        p_hbm.at[c, pl.ds(r0, rows_per_tile)])

    mesh = plsc.VectorSubcoreMesh(core_axis_name="c", subcore_axis_name="s")
    return pl.kernel(
        body,
        out_type=jax.ShapeDtypeStruct((_NC, n_pad, d), jnp.float32),
        mesh=mesh,
        scratch_types=[
            pltpu.VMEM((_CH,), jnp.int32),
            pltpu.VMEM((_CH,), jnp.int32),
            pltpu.VMEM((_CH,), jnp.int32),
            pltpu.VMEM((_CH,), jnp.int32),
            pltpu.VMEM((_CH, d), jnp.float32),
            pltpu.VMEM((_CH, d), jnp.float32),
            pltpu.VMEM_SHARED((n_pad, d), jnp.float32),
            pltpu.SemaphoreType.DMA,
            pltpu.SemaphoreType.DMA,
        ],
    )


def _make_sc_cnt(n_pad, d, e_pad):
    rows_per_tile = n_pad // _NS
    ept = e_pad // (_NC * _NS)
    nch = ept // _CH

    def body(dst_hbm, zrow_hbm, ones_hbm, cnt_hbm, dst_v, ones_v, cnt_sh):
        c = lax.axis_index("c")
        s = lax.axis_index("s")
        t = c * _NS + s
        r0 = s * rows_per_tile
        pltpu.sync_copy(zrow_hbm, cnt_sh.at[pl.ds(r0, rows_per_tile)])
        pltpu.sync_copy(dst_hbm.at[pl.ds(t * nch, nch)], dst_v)
        pltpu.sync_copy(ones_hbm, ones_v)
        plsc.subcore_barrier()

        def step(i, carry):
            pltpu.sync_copy(ones_v, cnt_sh.at[dst_v.at[i]], add=True)
            return carry

        lax.fori_loop(0, nch, step, 0)
        plsc.subcore_barrier()
        pltpu.sync_copy(cnt_sh.at[pl.ds(r0, rows_per_tile)],
                        cnt_hbm.at[c, pl.ds(r0, rows_per_tile)])

    mesh = plsc.VectorSubcoreMesh(core_axis_name="c", subcore_axis_name="s")
    return pl.kernel(
        body,
        out_type=jax.ShapeDtypeStruct((_NC, n_pad, d), jnp.float32),
        mesh=mesh,
        scratch_types=[
            pltpu.VMEM((ept // _CH, _CH), jnp.int32),
            pltpu.VMEM((_CH, d), jnp.float32),
            pltpu.VMEM_SHARED((n_pad, d), jnp.float32),
        ],
    )


def _layer_math(p, cnth, h, wl, bl, wr, g, b):
    agg = p[0] + p[1]
    cnt = cnth[0, :, 0] + cnth[1, :, 0]
    mean = agg / jnp.maximum(cnt, 1.0)[:, None]
    out = lax.dot_general(mean, wl, (((1,), (1,)), ((), ())),
                          preferred_element_type=jnp.float32) + bl[None, :]
    out = out + lax.dot_general(h, wr, (((1,), (1,)), ((), ())),
                                preferred_element_type=jnp.float32)
    mu = jnp.mean(out, axis=-1, keepdims=True)
    var = jnp.mean((out - mu) ** 2, axis=-1, keepdims=True)
    y = (out - mu) * lax.rsqrt(var + 1e-5) * g[None, :] + b[None, :]
    return jnp.maximum(y, 0.0)


def _tc_layer_body(p_ref, cnt_ref, h_ref, wl_ref, bl_ref, wr_ref, g_ref, b_ref,
                   o_ref):
    o_ref[...] = _layer_math(p_ref[...], cnt_ref[...], h_ref[...], wl_ref[...],
                             bl_ref[...], wr_ref[...], g_ref[...], b_ref[...])


def _tc_final_body(p_ref, cnt_ref, h_ref, wl_ref, bl_ref, wr_ref, g_ref, b_ref,
                   wo_ref, bo_ref, o_ref):
    hr = _layer_math(p_ref[...], cnt_ref[...], h_ref[...], wl_ref[...],
                     bl_ref[...], wr_ref[...], g_ref[...], b_ref[...])
    logits = lax.dot_general(hr, wo_ref[...], (((1,), (1,)), ((), ())),
                             preferred_element_type=jnp.float32) + bo_ref[...][None, :]
    m = jnp.max(logits, axis=-1, keepdims=True)
    lse = jnp.log(jnp.sum(jnp.exp(logits - m), axis=-1, keepdims=True)) + m
    o_ref[...] = logits - lse


_BLK = 400


def _make_tc_layer(n, d):
    grid = (n // _BLK,)
    in_specs = [
        pl.BlockSpec((_NC, _BLK, d), lambda i: (0, i, 0)),
        pl.BlockSpec((_NC, _BLK, d), lambda i: (0, i, 0)),
        pl.BlockSpec((_BLK, d), lambda i: (i, 0)),
        pl.BlockSpec((d, d), lambda i: (0, 0)),
        pl.BlockSpec((d,), lambda i: (0,)),
        pl.BlockSpec((d, d), lambda i: (0, 0)),
        pl.BlockSpec((d,), lambda i: (0,)),
        pl.BlockSpec((d,), lambda i: (0,)),
    ]
    return pl.pallas_call(
        _tc_layer_body,
        grid=grid,
        in_specs=in_specs,
        out_specs=pl.BlockSpec((_BLK, d), lambda i: (i, 0)),
        out_shape=jax.ShapeDtypeStruct((n, d), jnp.float32),
    )


def _make_tc_final(n, d, c_out):
    grid = (n // _BLK,)
    in_specs = [
        pl.BlockSpec((_NC, _BLK, d), lambda i: (0, i, 0)),
        pl.BlockSpec((_NC, _BLK, d), lambda i: (0, i, 0)),
        pl.BlockSpec((_BLK, d), lambda i: (i, 0)),
        pl.BlockSpec((d, d), lambda i: (0, 0)),
        pl.BlockSpec((d,), lambda i: (0,)),
        pl.BlockSpec((d, d), lambda i: (0, 0)),
        pl.BlockSpec((d,), lambda i: (0,)),
        pl.BlockSpec((d,), lambda i: (0,)),
        pl.BlockSpec((c_out, d), lambda i: (0, 0)),
        pl.BlockSpec((c_out,), lambda i: (0,)),
    ]
    return pl.pallas_call(
        _tc_final_body,
        grid=grid,
        in_specs=in_specs,
        out_specs=pl.BlockSpec((_BLK, c_out), lambda i: (i, 0)),
        out_shape=jax.ShapeDtypeStruct((n, c_out), jnp.float32),
    )


def kernel(x, edge_index, Wl0, bl0, Wr0, g0, b0, Wl1, bl1, Wr1, g1, b1,
           Wl2, bl2, Wr2, g2, b2, Wout, bout):
    n, d = x.shape
    e = edge_index.shape[1]
    c_out = Wout.shape[0]
    n_pad = ((n + _NS * 8 - 1) // (_NS * 8)) * (_NS * 8)
    unit = _NC * _NS * _CH * 2
    e_pad = ((e + unit - 1) // unit) * unit
    assert n < n_pad, "edge padding needs a spare accumulator row"

    dst = edge_index[0]
    src = edge_index[1]
    pad = e_pad - e
    src1 = jnp.concatenate([src, jnp.zeros((pad,), jnp.int32)])
    dst1 = jnp.concatenate([dst, jnp.full((pad,), n, jnp.int32)])
    dst2 = dst1.reshape(-1, _CH)
    zrow = jnp.zeros((n_pad // _NS, d), jnp.float32)
    ones = jnp.ones((_CH, d), jnp.float32)

    sc_agg = _make_sc_agg(n_pad, d, e_pad)
    sc_cnt = _make_sc_cnt(n_pad, d, e_pad)
    tc_layer = _make_tc_layer(n, d)
    tc_final = _make_tc_final(n, d, c_out)

    cnth = sc_cnt(dst2, zrow, ones)
    p0 = sc_agg(x, src1, dst1, zrow)
    h1 = tc_layer(p0, cnth, x, Wl0, bl0, Wr0, g0, b0)
    p1 = sc_agg(h1, src1, dst1, zrow)
    h2 = tc_layer(p1, cnth, h1, Wl1, bl1, Wr1, g1, b1)
    p2 = sc_agg(h2, src1, dst1, zrow)
    return tc_final(p2, cnth, h2, Wl2, bl2, Wr2, g2, b2, Wout, bout)


# speedup vs baseline: 1.0493x; 1.0493x over previous
import jax
import jax.numpy as jnp
from jax import lax
from jax.experimental import pallas as pl
from jax.experimental.pallas import tpu as pltpu
from jax.experimental.pallas import tpu_sc as plsc

_NC = 2
_NS = 16
_CH = 128


def _make_sc_agg(n_pad, d, e_pad):
    rows_per_tile = n_pad // _NS
    ept = e_pad // (_NC * _NS)
    nch = ept // _CH
    assert ept % (2 * _CH) == 0 and rows_per_tile % 8 == 0

    def body(h_hbm, src_hbm, dst_hbm, zrow_hbm, p_hbm,
             src_c0, src_c1, dst_c0, dst_c1, buf0, buf1, agg_sh, sem0, sem1):
        c = lax.axis_index("c")
        s = lax.axis_index("s")
        t = c * _NS + s
        r0 = s * rows_per_tile
        base = t * ept
        pltpu.sync_copy(zrow_hbm, agg_sh.at[pl.ds(r0, rows_per_tile)])
        plsc.subcore_barrier()

        def fetch(i, sref, dref):
            pltpu.sync_copy(src_hbm.at[pl.ds(base + i * _CH, _CH)], sref)
            pltpu.sync_copy(dst_hbm.at[pl.ds(base + i * _CH, _CH)], dref)

        def step(k, carry):
            i = 2 * k
            fetch(i, src_c0, dst_c0)
            pltpu.sync_copy(h_hbm.at[src_c0], buf0)
            pltpu.sync_copy(buf0, agg_sh.at[dst_c0], add=True)
            fetch(i + 1, src_c1, dst_c1)
            pltpu.sync_copy(h_hbm.at[src_c1], buf1)
            pltpu.sync_copy(buf1, agg_sh.at[dst_c1], add=True)
            return carry

        lax.fori_loop(0, nch // 2, step, 0)

        plsc.subcore_barrier()
        pltpu.sync_copy(agg_sh.at[pl.ds(r0, rows_per_tile)],
                        p_hbm.at[c, pl.ds(r0, rows_per_tile)])

    mesh = plsc.VectorSubcoreMesh(core_axis_name="c", subcore_axis_name="s")
    return pl.kernel(
        body,
        out_type=jax.ShapeDtypeStruct((_NC, n_pad, d), jnp.float32),
        mesh=mesh,
        scratch_types=[
            pltpu.VMEM((_CH,), jnp.int32),
            pltpu.VMEM((_CH,), jnp.int32),
            pltpu.VMEM((_CH,), jnp.int32),
            pltpu.VMEM((_CH,), jnp.int32),
            pltpu.VMEM((_CH, d), jnp.float32),
            pltpu.VMEM((_CH, d), jnp.float32),
            pltpu.VMEM_SHARED((n_pad, d), jnp.float32),
            pltpu.SemaphoreType.DMA,
            pltpu.SemaphoreType.DMA,
        ],
    )


def _make_sc_cnt(n_pad, d, e_pad):
    rows_per_tile = n_pad // _NS
    ept = e_pad // (_NC * _NS)
    nch = ept // _CH

    def body(dst_hbm, zrow_hbm, ones_hbm, cnt_hbm, dst_v, ones_v, cnt_sh):
        c = lax.axis_index("c")
        s = lax.axis_index("s")
        t = c * _NS + s
        r0 = s * rows_per_tile
        pltpu.sync_copy(zrow_hbm, cnt_sh.at[pl.ds(r0, rows_per_tile)])
        pltpu.sync_copy(dst_hbm.at[pl.ds(t * nch, nch)], dst_v)
        pltpu.sync_copy(ones_hbm, ones_v)
        plsc.subcore_barrier()

        def step(i, carry):
            pltpu.sync_copy(ones_v, cnt_sh.at[dst_v.at[i]], add=True)
            return carry

        lax.fori_loop(0, nch, step, 0)
        plsc.subcore_barrier()
        pltpu.sync_copy(cnt_sh.at[pl.ds(r0, rows_per_tile)],
                        cnt_hbm.at[c, pl.ds(r0, rows_per_tile)])

    mesh = plsc.VectorSubcoreMesh(core_axis_name="c", subcore_axis_name="s")
    return pl.kernel(
        body,
        out_type=jax.ShapeDtypeStruct((_NC, n_pad, d), jnp.float32),
        mesh=mesh,
        scratch_types=[
            pltpu.VMEM((ept // _CH, _CH), jnp.int32),
            pltpu.VMEM((_CH, d), jnp.float32),
            pltpu.VMEM_SHARED((n_pad, d), jnp.float32),
        ],
    )


def _layer_math(p, cnth, h, wl, bl, wr, g, b):
    agg = p[0] + p[1]
    cnt = cnth[0, :, 0] + cnth[1, :, 0]
    mean = agg / jnp.maximum(cnt, 1.0)[:, None]
    out = lax.dot_general(mean, wl, (((1,), (1,)), ((), ())),
                          preferred_element_type=jnp.float32) + bl[None, :]
    out = out + lax.dot_general(h, wr, (((1,), (1,)), ((), ())),
                                preferred_element_type=jnp.float32)
    mu = jnp.mean(out, axis=-1, keepdims=True)
    var = jnp.mean((out - mu) ** 2, axis=-1, keepdims=True)
    y = (out - mu) * lax.rsqrt(var + 1e-5) * g[None, :] + b[None, :]
    return jnp.maximum(y, 0.0)


def _tc_layer_body(p_ref, cnt_ref, h_ref, wl_ref, bl_ref, wr_ref, g_ref, b_ref,
                   o_ref):
    o_ref[...] = _layer_math(p_ref[...], cnt_ref[...], h_ref[...], wl_ref[...],
                             bl_ref[...], wr_ref[...], g_ref[...], b_ref[...])


def _tc_final_body(p_ref, cnt_ref, h_ref, wl_ref, bl_ref, wr_ref, g_ref, b_ref,
                   wo_ref, bo_ref, o_ref):
    hr = _layer_math(p_ref[...], cnt_ref[...], h_ref[...], wl_ref[...],
                     bl_ref[...], wr_ref[...], g_ref[...], b_ref[...])
    logits = lax.dot_general(hr, wo_ref[...], (((1,), (1,)), ((), ())),
                             preferred_element_type=jnp.float32) + bo_ref[...][None, :]
    m = jnp.max(logits, axis=-1, keepdims=True)
    lse = jnp.log(jnp.sum(jnp.exp(logits - m), axis=-1, keepdims=True)) + m
    o_ref[...] = logits - lse


_BLK = 400


def _make_tc_layer(n, d):
    grid = (n // _BLK,)
    in_specs = [
        pl.BlockSpec((_NC, _BLK, d), lambda i: (0, i, 0)),
        pl.BlockSpec((_NC, _BLK, d), lambda i: (0, i, 0)),
        pl.BlockSpec((_BLK, d), lambda i: (i, 0)),
        pl.BlockSpec((d, d), lambda i: (0, 0)),
        pl.BlockSpec((d,), lambda i: (0,)),
        pl.BlockSpec((d, d), lambda i: (0, 0)),
        pl.BlockSpec((d,), lambda i: (0,)),
        pl.BlockSpec((d,), lambda i: (0,)),
    ]
    return pl.pallas_call(
        _tc_layer_body,
        grid=grid,
        in_specs=in_specs,
        out_specs=pl.BlockSpec((_BLK, d), lambda i: (i, 0)),
        out_shape=jax.ShapeDtypeStruct((n, d), jnp.float32),
    )


def _make_tc_final(n, d, c_out):
    grid = (n // _BLK,)
    in_specs = [
        pl.BlockSpec((_NC, _BLK, d), lambda i: (0, i, 0)),
        pl.BlockSpec((_NC, _BLK, d), lambda i: (0, i, 0)),
        pl.BlockSpec((_BLK, d), lambda i: (i, 0)),
        pl.BlockSpec((d, d), lambda i: (0, 0)),
        pl.BlockSpec((d,), lambda i: (0,)),
        pl.BlockSpec((d, d), lambda i: (0, 0)),
        pl.BlockSpec((d,), lambda i: (0,)),
        pl.BlockSpec((d,), lambda i: (0,)),
        pl.BlockSpec((c_out, d), lambda i: (0, 0)),
        pl.BlockSpec((c_out,), lambda i: (0,)),
    ]
    return pl.pallas_call(
        _tc_final_body,
        grid=grid,
        in_specs=in_specs,
        out_specs=pl.BlockSpec((_BLK, c_out), lambda i: (i, 0)),
        out_shape=jax.ShapeDtypeStruct((n, c_out), jnp.float32),
    )


def kernel(x, edge_index, Wl0, bl0, Wr0, g0, b0, Wl1, bl1, Wr1, g1, b1,
           Wl2, bl2, Wr2, g2, b2, Wout, bout):
    n, d = x.shape
    e = edge_index.shape[1]
    c_out = Wout.shape[0]
    n_pad = ((n + _NS * 8 - 1) // (_NS * 8)) * (_NS * 8)
    unit = _NC * _NS * _CH * 2
    e_pad = ((e + unit - 1) // unit) * unit
    assert n < n_pad, "edge padding needs a spare accumulator row"

    dst = edge_index[0]
    src = edge_index[1]
    pad = e_pad - e
    src1 = jnp.concatenate([src, jnp.zeros((pad,), jnp.int32)])
    pad_dst = n + jnp.arange(pad, dtype=jnp.int32) % (n_pad - n)
    dst1 = jnp.concatenate([dst, pad_dst])
    dst2 = dst1.reshape(-1, _CH)
    zrow = jnp.zeros((n_pad // _NS, d), jnp.float32)
    ones = jnp.ones((_CH, d), jnp.float32)

    sc_agg = _make_sc_agg(n_pad, d, e_pad)
    sc_cnt = _make_sc_cnt(n_pad, d, e_pad)
    tc_layer = _make_tc_layer(n, d)
    tc_final = _make_tc_final(n, d, c_out)

    cnth = sc_cnt(dst2, zrow, ones)
    p0 = sc_agg(x, src1, dst1, zrow)
    h1 = tc_layer(p0, cnth, x, Wl0, bl0, Wr0, g0, b0)
    p1 = sc_agg(h1, src1, dst1, zrow)
    h2 = tc_layer(p1, cnth, h1, Wl1, bl1, Wr1, g1, b1)
    p2 = sc_agg(h2, src1, dst1, zrow)
    return tc_final(p2, cnth, h2, Wl2, bl2, Wr2, g2, b2, Wout, bout)


# speedup vs baseline: 1.3620x; 1.2981x over previous
import jax
import jax.numpy as jnp
from jax import lax
from jax.experimental import pallas as pl
from jax.experimental.pallas import tpu as pltpu
from jax.experimental.pallas import tpu_sc as plsc

_NC = 2
_NS = 16
_CH = 80


def _make_sc_agg(n_pad, d, e_pad):
    rows_per_tile = n_pad // _NS
    ept = e_pad // (_NC * _NS)
    nch = ept // _CH
    assert ept % (2 * _CH) == 0 and rows_per_tile % 8 == 0

    def body(h_hbm, src_hbm, dst_hbm, zrow_hbm, p_hbm,
             src_c0, src_c1, dst_c0, dst_c1, buf0, buf1, agg_sh, sem0, sem1):
        c = lax.axis_index("c")
        s = lax.axis_index("s")
        t = c * _NS + s
        r0 = s * rows_per_tile
        base = t * ept
        pltpu.sync_copy(zrow_hbm, agg_sh.at[pl.ds(r0, rows_per_tile)])
        plsc.subcore_barrier()

        def fetch(i, sref, dref):
            pltpu.sync_copy(src_hbm.at[pl.ds(base + i * _CH, _CH)], sref)
            pltpu.sync_copy(dst_hbm.at[pl.ds(base + i * _CH, _CH)], dref)

        def step(k, carry):
            i = 2 * k
            fetch(i, src_c0, dst_c0)
            pltpu.sync_copy(h_hbm.at[src_c0], buf0)
            pltpu.sync_copy(buf0, agg_sh.at[dst_c0], add=True)
            fetch(i + 1, src_c1, dst_c1)
            pltpu.sync_copy(h_hbm.at[src_c1], buf1)
            pltpu.sync_copy(buf1, agg_sh.at[dst_c1], add=True)
            return carry

        lax.fori_loop(0, nch // 2, step, 0)

        plsc.subcore_barrier()
        pltpu.sync_copy(agg_sh.at[pl.ds(r0, rows_per_tile)],
                        p_hbm.at[c, pl.ds(r0, rows_per_tile)])

    mesh = plsc.VectorSubcoreMesh(core_axis_name="c", subcore_axis_name="s")
    return pl.kernel(
        body,
        out_type=jax.ShapeDtypeStruct((_NC, n_pad, d), jnp.float32),
        mesh=mesh,
        scratch_types=[
            pltpu.VMEM((_CH,), jnp.int32),
            pltpu.VMEM((_CH,), jnp.int32),
            pltpu.VMEM((_CH,), jnp.int32),
            pltpu.VMEM((_CH,), jnp.int32),
            pltpu.VMEM((_CH, d), jnp.float32),
            pltpu.VMEM((_CH, d), jnp.float32),
            pltpu.VMEM_SHARED((n_pad, d), jnp.float32),
            pltpu.SemaphoreType.DMA,
            pltpu.SemaphoreType.DMA,
        ],
    )


def _make_sc_cnt(n_pad, d, e_pad):
    rows_per_tile = n_pad // _NS
    ept = e_pad // (_NC * _NS)
    nch = ept // _CH

    def body(dst_hbm, zrow_hbm, ones_hbm, cnt_hbm, dst_c, ones_v, cnt_sh):
        c = lax.axis_index("c")
        s = lax.axis_index("s")
        t = c * _NS + s
        r0 = s * rows_per_tile
        base = t * ept
        pltpu.sync_copy(zrow_hbm, cnt_sh.at[pl.ds(r0, rows_per_tile)])
        pltpu.sync_copy(ones_hbm, ones_v)
        plsc.subcore_barrier()

        def step(i, carry):
            pltpu.sync_copy(dst_hbm.at[pl.ds(base + i * _CH, _CH)], dst_c)
            pltpu.sync_copy(ones_v, cnt_sh.at[dst_c], add=True)
            return carry

        lax.fori_loop(0, nch, step, 0)
        plsc.subcore_barrier()
        pltpu.sync_copy(cnt_sh.at[pl.ds(r0, rows_per_tile)],
                        cnt_hbm.at[c, pl.ds(r0, rows_per_tile)])

    mesh = plsc.VectorSubcoreMesh(core_axis_name="c", subcore_axis_name="s")
    return pl.kernel(
        body,
        out_type=jax.ShapeDtypeStruct((_NC, n_pad, d), jnp.float32),
        mesh=mesh,
        scratch_types=[
            pltpu.VMEM((_CH,), jnp.int32),
            pltpu.VMEM((_CH, d), jnp.float32),
            pltpu.VMEM_SHARED((n_pad, d), jnp.float32),
        ],
    )


def _layer_math(p, cnth, h, wl, bl, wr, g, b):
    agg = p[0] + p[1]
    cnt = cnth[0, :, 0] + cnth[1, :, 0]
    mean = agg / jnp.maximum(cnt, 1.0)[:, None]
    out = lax.dot_general(mean, wl, (((1,), (1,)), ((), ())),
                          preferred_element_type=jnp.float32) + bl[None, :]
    out = out + lax.dot_general(h, wr, (((1,), (1,)), ((), ())),
                                preferred_element_type=jnp.float32)
    mu = jnp.mean(out, axis=-1, keepdims=True)
    var = jnp.mean((out - mu) ** 2, axis=-1, keepdims=True)
    y = (out - mu) * lax.rsqrt(var + 1e-5) * g[None, :] + b[None, :]
    return jnp.maximum(y, 0.0)


def _tc_layer_body(p_ref, cnt_ref, h_ref, wl_ref, bl_ref, wr_ref, g_ref, b_ref,
                   o_ref):
    o_ref[...] = _layer_math(p_ref[...], cnt_ref[...], h_ref[...], wl_ref[...],
                             bl_ref[...], wr_ref[...], g_ref[...], b_ref[...])


def _tc_final_body(p_ref, cnt_ref, h_ref, wl_ref, bl_ref, wr_ref, g_ref, b_ref,
                   wo_ref, bo_ref, o_ref):
    hr = _layer_math(p_ref[...], cnt_ref[...], h_ref[...], wl_ref[...],
                     bl_ref[...], wr_ref[...], g_ref[...], b_ref[...])
    logits = lax.dot_general(hr, wo_ref[...], (((1,), (1,)), ((), ())),
                             preferred_element_type=jnp.float32) + bo_ref[...][None, :]
    m = jnp.max(logits, axis=-1, keepdims=True)
    lse = jnp.log(jnp.sum(jnp.exp(logits - m), axis=-1, keepdims=True)) + m
    o_ref[...] = logits - lse


_BLK = 400


def _make_tc_layer(n, d):
    grid = (n // _BLK,)
    in_specs = [
        pl.BlockSpec((_NC, _BLK, d), lambda i: (0, i, 0)),
        pl.BlockSpec((_NC, _BLK, d), lambda i: (0, i, 0)),
        pl.BlockSpec((_BLK, d), lambda i: (i, 0)),
        pl.BlockSpec((d, d), lambda i: (0, 0)),
        pl.BlockSpec((d,), lambda i: (0,)),
        pl.BlockSpec((d, d), lambda i: (0, 0)),
        pl.BlockSpec((d,), lambda i: (0,)),
        pl.BlockSpec((d,), lambda i: (0,)),
    ]
    return pl.pallas_call(
        _tc_layer_body,
        grid=grid,
        in_specs=in_specs,
        out_specs=pl.BlockSpec((_BLK, d), lambda i: (i, 0)),
        out_shape=jax.ShapeDtypeStruct((n, d), jnp.float32),
    )


def _make_tc_final(n, d, c_out):
    grid = (n // _BLK,)
    in_specs = [
        pl.BlockSpec((_NC, _BLK, d), lambda i: (0, i, 0)),
        pl.BlockSpec((_NC, _BLK, d), lambda i: (0, i, 0)),
        pl.BlockSpec((_BLK, d), lambda i: (i, 0)),
        pl.BlockSpec((d, d), lambda i: (0, 0)),
        pl.BlockSpec((d,), lambda i: (0,)),
        pl.BlockSpec((d, d), lambda i: (0, 0)),
        pl.BlockSpec((d,), lambda i: (0,)),
        pl.BlockSpec((d,), lambda i: (0,)),
        pl.BlockSpec((c_out, d), lambda i: (0, 0)),
        pl.BlockSpec((c_out,), lambda i: (0,)),
    ]
    return pl.pallas_call(
        _tc_final_body,
        grid=grid,
        in_specs=in_specs,
        out_specs=pl.BlockSpec((_BLK, c_out), lambda i: (i, 0)),
        out_shape=jax.ShapeDtypeStruct((n, c_out), jnp.float32),
    )


def kernel(x, edge_index, Wl0, bl0, Wr0, g0, b0, Wl1, bl1, Wr1, g1, b1,
           Wl2, bl2, Wr2, g2, b2, Wout, bout):
    n, d = x.shape
    e = edge_index.shape[1]
    c_out = Wout.shape[0]
    n_pad = ((n + _NS * 8 - 1) // (_NS * 8)) * (_NS * 8)
    unit = _NC * _NS * _CH * 2
    e_pad = ((e + unit - 1) // unit) * unit
    assert n < n_pad, "edge padding needs a spare accumulator row"

    dst = edge_index[0]
    src = edge_index[1]
    pad = e_pad - e
    src1 = jnp.concatenate([src, jnp.zeros((pad,), jnp.int32)])
    pad_dst = n + jnp.arange(pad, dtype=jnp.int32) % (n_pad - n)
    dst1 = jnp.concatenate([dst, pad_dst])
    zrow = jnp.zeros((n_pad // _NS, d), jnp.float32)
    ones = jnp.ones((_CH, d), jnp.float32)

    sc_agg = _make_sc_agg(n_pad, d, e_pad)
    sc_cnt = _make_sc_cnt(n_pad, d, e_pad)
    tc_layer = _make_tc_layer(n, d)
    tc_final = _make_tc_final(n, d, c_out)

    cnth = sc_cnt(dst1, zrow, ones)
    p0 = sc_agg(x, src1, dst1, zrow)
    h1 = tc_layer(p0, cnth, x, Wl0, bl0, Wr0, g0, b0)
    p1 = sc_agg(h1, src1, dst1, zrow)
    h2 = tc_layer(p1, cnth, h1, Wl1, bl1, Wr1, g1, b1)
    p2 = sc_agg(h2, src1, dst1, zrow)
    return tc_final(p2, cnth, h2, Wl2, bl2, Wr2, g2, b2, Wout, bout)


# speedup vs baseline: 2.1310x; 1.5645x over previous
import jax
import jax.numpy as jnp
from jax import lax
from jax.experimental import pallas as pl
from jax.experimental.pallas import tpu as pltpu
from jax.experimental.pallas import tpu_sc as plsc

_NC = 2
_NS = 16
_CH = 80


def _make_sc_agg(n_pad, d, e_pad):
    rows_per_tile = n_pad // _NS
    ept = e_pad // (_NC * _NS)
    nch = ept // _CH
    assert ept % (2 * _CH) == 0 and rows_per_tile % 8 == 0

    def body(h_hbm, sd_hbm, zrow_hbm, p_hbm,
             sd0, sd1, buf0, buf1, agg_sh, gsem0, gsem1, ssem0, ssem1):
        c = lax.axis_index("c")
        s = lax.axis_index("s")
        t = c * _NS + s
        r0 = s * rows_per_tile
        cbase = t * nch
        pltpu.sync_copy(zrow_hbm, agg_sh.at[pl.ds(r0, rows_per_tile)])
        plsc.subcore_barrier()

        def drain(sem, buf):
            pltpu.make_async_copy(h_hbm.at[pl.ds(0, _CH)], buf, sem).wait()

        pltpu.sync_copy(sd_hbm.at[cbase], sd0)
        pltpu.async_copy(h_hbm.at[sd0.at[0]], buf0, gsem0)
        pltpu.sync_copy(sd_hbm.at[cbase + 1], sd1)
        pltpu.async_copy(h_hbm.at[sd1.at[0]], buf1, gsem1)

        def step(k, carry):
            i = 2 * k
            drain(gsem0, buf0)
            pltpu.async_copy(buf0, agg_sh.at[sd0.at[1]], ssem0, add=True)
            drain(ssem0, buf0)
            pltpu.sync_copy(sd_hbm.at[cbase + i + 2], sd0)
            pltpu.async_copy(h_hbm.at[sd0.at[0]], buf0, gsem0)
            drain(gsem1, buf1)
            pltpu.async_copy(buf1, agg_sh.at[sd1.at[1]], ssem1, add=True)
            drain(ssem1, buf1)
            pltpu.sync_copy(sd_hbm.at[cbase + i + 3], sd1)
            pltpu.async_copy(h_hbm.at[sd1.at[0]], buf1, gsem1)
            return carry

        lax.fori_loop(0, nch // 2 - 1, step, 0)
        drain(gsem0, buf0)
        pltpu.async_copy(buf0, agg_sh.at[sd0.at[1]], ssem0, add=True)
        drain(gsem1, buf1)
        pltpu.async_copy(buf1, agg_sh.at[sd1.at[1]], ssem1, add=True)
        drain(ssem0, buf0)
        drain(ssem1, buf1)

        plsc.subcore_barrier()
        pltpu.sync_copy(agg_sh.at[pl.ds(r0, rows_per_tile)],
                        p_hbm.at[c, pl.ds(r0, rows_per_tile)])

    mesh = plsc.VectorSubcoreMesh(core_axis_name="c", subcore_axis_name="s")
    return pl.kernel(
        body,
        out_type=jax.ShapeDtypeStruct((_NC, n_pad, d), jnp.float32),
        mesh=mesh,
        scratch_types=[
            pltpu.VMEM((2, _CH), jnp.int32),
            pltpu.VMEM((2, _CH), jnp.int32),
            pltpu.VMEM((_CH, d), jnp.float32),
            pltpu.VMEM((_CH, d), jnp.float32),
            pltpu.VMEM_SHARED((n_pad, d), jnp.float32),
            pltpu.SemaphoreType.DMA,
            pltpu.SemaphoreType.DMA,
            pltpu.SemaphoreType.DMA,
            pltpu.SemaphoreType.DMA,
        ],
    )


def _make_sc_cnt(n_pad, d, e_pad):
    rows_per_tile = n_pad // _NS
    ept = e_pad // (_NC * _NS)
    nch = ept // _CH

    def body(dst_hbm, zrow_hbm, ones_hbm, cnt_hbm,
             dst_c0, dst_c1, ones_v, cnt_sh, ssem0, ssem1):
        c = lax.axis_index("c")
        s = lax.axis_index("s")
        t = c * _NS + s
        r0 = s * rows_per_tile
        base = t * ept
        pltpu.sync_copy(zrow_hbm, cnt_sh.at[pl.ds(r0, rows_per_tile)])
        pltpu.sync_copy(ones_hbm, ones_v)
        plsc.subcore_barrier()

        def drain(sem):
            pltpu.make_async_copy(zrow_hbm.at[pl.ds(0, _CH)], ones_v, sem).wait()

        pltpu.sync_copy(dst_hbm.at[pl.ds(base, _CH)], dst_c0)
        pltpu.async_copy(ones_v, cnt_sh.at[dst_c0], ssem0, add=True)
        pltpu.sync_copy(dst_hbm.at[pl.ds(base + _CH, _CH)], dst_c1)
        pltpu.async_copy(ones_v, cnt_sh.at[dst_c1], ssem1, add=True)

        def step(k, carry):
            i = 2 * k + 2
            drain(ssem0)
            pltpu.sync_copy(dst_hbm.at[pl.ds(base + i * _CH, _CH)], dst_c0)
            pltpu.async_copy(ones_v, cnt_sh.at[dst_c0], ssem0, add=True)
            drain(ssem1)
            pltpu.sync_copy(dst_hbm.at[pl.ds(base + (i + 1) * _CH, _CH)], dst_c1)
            pltpu.async_copy(ones_v, cnt_sh.at[dst_c1], ssem1, add=True)
            return carry

        lax.fori_loop(0, nch // 2 - 1, step, 0)
        drain(ssem0)
        drain(ssem1)
        plsc.subcore_barrier()
        pltpu.sync_copy(cnt_sh.at[pl.ds(r0, rows_per_tile)],
                        cnt_hbm.at[c, pl.ds(r0, rows_per_tile)])

    mesh = plsc.VectorSubcoreMesh(core_axis_name="c", subcore_axis_name="s")
    return pl.kernel(
        body,
        out_type=jax.ShapeDtypeStruct((_NC, n_pad, d), jnp.float32),
        mesh=mesh,
        scratch_types=[
            pltpu.VMEM((_CH,), jnp.int32),
            pltpu.VMEM((_CH,), jnp.int32),
            pltpu.VMEM((_CH, d), jnp.float32),
            pltpu.VMEM_SHARED((n_pad, d), jnp.float32),
            pltpu.SemaphoreType.DMA,
            pltpu.SemaphoreType.DMA,
        ],
    )


def _layer_math(p, cnth, h, wl, bl, wr, g, b):
    agg = p[0] + p[1]
    cnt = cnth[0, :, 0] + cnth[1, :, 0]
    mean = agg / jnp.maximum(cnt, 1.0)[:, None]
    out = lax.dot_general(mean, wl, (((1,), (1,)), ((), ())),
                          preferred_element_type=jnp.float32) + bl[None, :]
    out = out + lax.dot_general(h, wr, (((1,), (1,)), ((), ())),
                                preferred_element_type=jnp.float32)
    mu = jnp.mean(out, axis=-1, keepdims=True)
    var = jnp.mean((out - mu) ** 2, axis=-1, keepdims=True)
    y = (out - mu) * lax.rsqrt(var + 1e-5) * g[None, :] + b[None, :]
    return jnp.maximum(y, 0.0)


def _tc_layer_body(p_ref, cnt_ref, h_ref, wl_ref, bl_ref, wr_ref, g_ref, b_ref,
                   o_ref):
    o_ref[...] = _layer_math(p_ref[...], cnt_ref[...], h_ref[...], wl_ref[...],
                             bl_ref[...], wr_ref[...], g_ref[...], b_ref[...])


def _tc_final_body(p_ref, cnt_ref, h_ref, wl_ref, bl_ref, wr_ref, g_ref, b_ref,
                   wo_ref, bo_ref, o_ref):
    hr = _layer_math(p_ref[...], cnt_ref[...], h_ref[...], wl_ref[...],
                     bl_ref[...], wr_ref[...], g_ref[...], b_ref[...])
    logits = lax.dot_general(hr, wo_ref[...], (((1,), (1,)), ((), ())),
                             preferred_element_type=jnp.float32) + bo_ref[...][None, :]
    m = jnp.max(logits, axis=-1, keepdims=True)
    lse = jnp.log(jnp.sum(jnp.exp(logits - m), axis=-1, keepdims=True)) + m
    o_ref[...] = logits - lse


_BLK = 400


def _make_tc_layer(n, d):
    grid = (n // _BLK,)
    in_specs = [
        pl.BlockSpec((_NC, _BLK, d), lambda i: (0, i, 0)),
        pl.BlockSpec((_NC, _BLK, d), lambda i: (0, i, 0)),
        pl.BlockSpec((_BLK, d), lambda i: (i, 0)),
        pl.BlockSpec((d, d), lambda i: (0, 0)),
        pl.BlockSpec((d,), lambda i: (0,)),
        pl.BlockSpec((d, d), lambda i: (0, 0)),
        pl.BlockSpec((d,), lambda i: (0,)),
        pl.BlockSpec((d,), lambda i: (0,)),
    ]
    return pl.pallas_call(
        _tc_layer_body,
        grid=grid,
        in_specs=in_specs,
        out_specs=pl.BlockSpec((_BLK, d), lambda i: (i, 0)),
        out_shape=jax.ShapeDtypeStruct((n, d), jnp.float32),
    )


def _make_tc_final(n, d, c_out):
    grid = (n // _BLK,)
    in_specs = [
        pl.BlockSpec((_NC, _BLK, d), lambda i: (0, i, 0)),
        pl.BlockSpec((_NC, _BLK, d), lambda i: (0, i, 0)),
        pl.BlockSpec((_BLK, d), lambda i: (i, 0)),
        pl.BlockSpec((d, d), lambda i: (0, 0)),
        pl.BlockSpec((d,), lambda i: (0,)),
        pl.BlockSpec((d, d), lambda i: (0, 0)),
        pl.BlockSpec((d,), lambda i: (0,)),
        pl.BlockSpec((d,), lambda i: (0,)),
        pl.BlockSpec((c_out, d), lambda i: (0, 0)),
        pl.BlockSpec((c_out,), lambda i: (0,)),
    ]
    return pl.pallas_call(
        _tc_final_body,
        grid=grid,
        in_specs=in_specs,
        out_specs=pl.BlockSpec((_BLK, c_out), lambda i: (i, 0)),
        out_shape=jax.ShapeDtypeStruct((n, c_out), jnp.float32),
    )


def kernel(x, edge_index, Wl0, bl0, Wr0, g0, b0, Wl1, bl1, Wr1, g1, b1,
           Wl2, bl2, Wr2, g2, b2, Wout, bout):
    n, d = x.shape
    e = edge_index.shape[1]
    c_out = Wout.shape[0]
    n_pad = ((n + _NS * 8 - 1) // (_NS * 8)) * (_NS * 8)
    unit = _NC * _NS * _CH * 2
    e_pad = ((e + unit - 1) // unit) * unit
    assert n < n_pad, "edge padding needs a spare accumulator row"

    dst = edge_index[0]
    src = edge_index[1]
    pad = e_pad - e
    src1 = jnp.concatenate([src, jnp.zeros((pad,), jnp.int32)])
    pad_dst = n + jnp.arange(pad, dtype=jnp.int32) % (n_pad - n)
    dst1 = jnp.concatenate([dst, pad_dst])
    sd = jnp.stack([src1.reshape(-1, _CH), dst1.reshape(-1, _CH)], axis=1)
    zrow = jnp.zeros((n_pad // _NS, d), jnp.float32)
    ones = jnp.ones((_CH, d), jnp.float32)

    sc_agg = _make_sc_agg(n_pad, d, e_pad)
    sc_cnt = _make_sc_cnt(n_pad, d, e_pad)
    tc_layer = _make_tc_layer(n, d)
    tc_final = _make_tc_final(n, d, c_out)

    cnth = sc_cnt(dst1, zrow, ones)
    p0 = sc_agg(x, sd, zrow)
    h1 = tc_layer(p0, cnth, x, Wl0, bl0, Wr0, g0, b0)
    p1 = sc_agg(h1, sd, zrow)
    h2 = tc_layer(p1, cnth, h1, Wl1, bl1, Wr1, g1, b1)
    p2 = sc_agg(h2, sd, zrow)
    return tc_final(p2, cnth, h2, Wl2, bl2, Wr2, g2, b2, Wout, bout)
